# baseline (speedup 1.0000x reference)
import jax
import jax.numpy as jnp
from jax import lax
from jax.experimental import pallas as pl
from jax.experimental.pallas import tpu as pltpu
from jax.experimental.pallas import tpu_sc as plsc

N = 10000
E = 320000
D = 128

NP = 10112
ROWS_PER_SUB = 632
EP = 327680
ER = EP // 128
CHUNKS = 80
ER_PER_CORE = ER // 2
SLOW_CORE = 1
SLOW_ROWS = 32
FAST_ROWS = 128

_vec_mesh = plsc.VectorSubcoreMesh(core_axis_name="c", subcore_axis_name="s")


def _sc_count(dstm, zeros128, ones128):
    @pl.kernel(
        out_type=jax.ShapeDtypeStruct((2, NP, 128), jnp.float32),
        mesh=_vec_mesh,
        scratch_types=[
            pltpu.VMEM((CHUNKS, 128), jnp.int32),
            pltpu.VMEM((128, 128), jnp.float32),
            pltpu.VMEM_SHARED((NP, 128), jnp.float32),
        ],
    )
    def count_kernel(dstm_hbm, z_hbm, ones_hbm, out_hbm, dst_v, ones_v, acc):
        c = lax.axis_index("c")
        s = lax.axis_index("s")
        r0 = s * ROWS_PER_SUB
        pltpu.sync_copy(z_hbm.at[pl.ds(r0, ROWS_PER_SUB)],
                        acc.at[pl.ds(r0, ROWS_PER_SUB)])
        pltpu.sync_copy(dstm_hbm.at[pl.ds(c * ER_PER_CORE + s * CHUNKS, CHUNKS)],
                        dst_v)
        pltpu.sync_copy(ones_hbm, ones_v)
        plsc.subcore_barrier()

        @pl.loop(0, CHUNKS)
        def _(j):
            pltpu.sync_copy(ones_v, acc.at[dst_v.at[j]], add=True)

        plsc.subcore_barrier()
        pltpu.sync_copy(acc.at[pl.ds(r0, ROWS_PER_SUB)],
                        out_hbm.at[c].at[pl.ds(r0, ROWS_PER_SUB)])

    return count_kernel(dstm, zeros128, ones128)


def _sc_aggregate(g, srcm, dstm, zeros128):
    @pl.kernel(
        out_type=jax.ShapeDtypeStruct((2, NP, D), jnp.float32),
        mesh=_vec_mesh,
        scratch_types=[
            pltpu.VMEM((CHUNKS // 2, 128), jnp.int32),
            pltpu.VMEM((CHUNKS // 2, 128), jnp.int32),
            pltpu.VMEM((64, D), jnp.float32),
            pltpu.VMEM((64, D), jnp.float32),
            pltpu.VMEM((64, D), jnp.float32),
            pltpu.VMEM((64, D), jnp.float32),
            pltpu.VMEM_SHARED((NP, D), jnp.float32),
            pltpu.SemaphoreType.DMA,
            pltpu.SemaphoreType.DMA,
            pltpu.SemaphoreType.DMA,
            pltpu.SemaphoreType.DMA,
        ],
    )
    def agg_kernel(g_hbm, srcm_hbm, dstm_hbm, z_hbm, out_hbm,
                   src_v, dst_v, rb0, rb1, rb2, rb3, acc, sm0, sm1, sm2, sm3):
        c = lax.axis_index("c")
        s = lax.axis_index("s")
        bufs = [rb0, rb1, rb2, rb3]
        sems = [sm0, sm1, sm2, sm3]
        r0 = s * ROWS_PER_SUB
        pltpu.sync_copy(z_hbm.at[pl.ds(r0, ROWS_PER_SUB)],
                        acc.at[pl.ds(r0, ROWS_PER_SUB)])
        plsc.subcore_barrier()

        def sidx(ref, q):
            return ref.at[q // 2, pl.ds(64 * (q % 2), 64)]

        def do_edges(base, stages):
            off = 0
            for nrows in stages:
                nsub = nrows * 2
                rsl = pl.ds(0, nrows)
                pltpu.sync_copy(srcm_hbm.at[pl.ds(base + off, nrows)],
                                src_v.at[rsl])
                pltpu.sync_copy(dstm_hbm.at[pl.ds(base + off, nrows)],
                                dst_v.at[rsl])
                for u in range(3):
                    pltpu.async_copy(g_hbm.at[sidx(src_v, u)], bufs[u], sems[u])

                @pl.loop(0, nsub // 4)
                def _(k):
                    for u in range(4):
                        q = k * 4 + u
                        pltpu.make_async_copy(g_hbm.at[sidx(src_v, q)],
                                              bufs[u], sems[u]).wait()

                        @pl.when(q + 3 < nsub)
                        def _():
                            pltpu.async_copy(g_hbm.at[sidx(src_v, q + 3)],
                                             bufs[(u + 3) % 4],
                                             sems[(u + 3) % 4])

                        pltpu.sync_copy(bufs[u], acc.at[sidx(dst_v, q)],
                                        add=True)

                off += nrows

        @pl.when(c == SLOW_CORE)
        def _():
            do_edges(s * SLOW_ROWS, [SLOW_ROWS])

        @pl.when(c != SLOW_CORE)
        def _():
            do_edges(16 * SLOW_ROWS + s * FAST_ROWS,
                     [FAST_ROWS // 4] * 4)

        plsc.subcore_barrier()
        pltpu.sync_copy(acc.at[pl.ds(r0, ROWS_PER_SUB)],
                        out_hbm.at[c].at[pl.ds(r0, ROWS_PER_SUB)])

    return agg_kernel(g, srcm, dstm, zeros128)


def _mm_body(x_ref, w_ref, o_ref):
    o_ref[...] = jnp.dot(x_ref[...], w_ref[...],
                         preferred_element_type=jnp.float32)


def _tc_matmul(x, w):
    return pl.pallas_call(
        _mm_body,
        out_shape=jax.ShapeDtypeStruct((NP, D), jnp.float32),
    )(x, w)


def _scale_body(c0_ref, c1_ref, h_ref, g_ref, dinv_ref):
    deg = c0_ref[:, 0:1] + c1_ref[:, 0:1] + 1.0
    rows = lax.broadcasted_iota(jnp.int32, (NP, 1), 0)
    dinv = jnp.where(rows < N, lax.rsqrt(deg), 0.0)
    dinv_ref[...] = dinv
    g_ref[...] = h_ref[...] * dinv


def _tc_scale(cnt0, cnt1, h):
    return pl.pallas_call(
        _scale_body,
        out_shape=(jax.ShapeDtypeStruct((NP, D), jnp.float32),
                   jax.ShapeDtypeStruct((NP, 1), jnp.float32)),
    )(cnt0, cnt1, h)


def _layer2_body(sa_ref, sb_ref, g1_ref, dinv_ref, b1_ref, w2_ref, g2_ref):
    dinv = dinv_ref[...]
    t = dinv * (sa_ref[...] + sb_ref[...] + g1_ref[...]) + b1_ref[...]
    z = jnp.maximum(t, 0.0)
    h2 = jnp.dot(z, w2_ref[...], preferred_element_type=jnp.float32)
    g2_ref[...] = h2 * dinv


def _tc_layer2(sa, sb, g1, dinv, b1, w2):
    return pl.pallas_call(
        _layer2_body,
        out_shape=jax.ShapeDtypeStruct((NP, D), jnp.float32),
    )(sa, sb, g1, dinv, b1, w2)


def _final_body(sa_ref, sb_ref, g2_ref, dinv_ref, b2_ref, o_ref):
    t = dinv_ref[...] * (sa_ref[...] + sb_ref[...] + g2_ref[...]) + b2_ref[...]
    m = jnp.max(t, axis=1, keepdims=True)
    lse = jnp.log(jnp.sum(jnp.exp(t - m), axis=1, keepdims=True)) + m
    o_ref[...] = t - lse


def _tc_final(sa, sb, g2, dinv, b2):
    return pl.pallas_call(
        _final_body,
        out_shape=jax.ShapeDtypeStruct((NP, D), jnp.float32),
    )(sa, sb, g2, dinv, b2)


def kernel(x, edge_index, W1, b1, W2, b2):
    f32 = jnp.float32
    xp = jnp.pad(x, ((0, NP - N), (0, 0)))
    pad = jnp.full((EP - E,), N, dtype=jnp.int32)
    srcm = jnp.concatenate([edge_index[0].astype(jnp.int32), pad]).reshape(ER, 128)
    dstm = jnp.concatenate([edge_index[1].astype(jnp.int32), pad]).reshape(ER, 128)
    zeros128 = jnp.zeros((NP, D), f32)
    ones128 = jnp.ones((128, 128), f32)

    cnt = _sc_count(dstm, zeros128, ones128)
    h1 = _tc_matmul(xp, W1)
    g1, dinv = _tc_scale(cnt[0], cnt[1], h1)
    s1 = _sc_aggregate(g1, srcm, dstm, zeros128)
    g2 = _tc_layer2(s1[0], s1[1], g1, dinv, b1.reshape(1, D), W2)
    s2 = _sc_aggregate(g2, srcm, dstm, zeros128)
    outp = _tc_final(s2[0], s2[1], g2, dinv, b2.reshape(1, D))
    return outp[:N]


# speedup vs baseline: 1.1988x; 1.1988x over previous
import jax
import jax.numpy as jnp
from jax import lax
from jax.experimental import pallas as pl
from jax.experimental.pallas import tpu as pltpu
from jax.experimental.pallas import tpu_sc as plsc

N = 10000
E = 320000
D = 128

NP = 10112
ROWS_PER_SUB = 632
EP = 327680
ER = EP // 128
CHUNKS = 80
ER_PER_CORE = ER // 2
SLOW_CORE = 1
SLOW_ROWS = 32
FAST_ROWS = 128

_vec_mesh = plsc.VectorSubcoreMesh(core_axis_name="c", subcore_axis_name="s")


def _sc_count(dstm, zeros128, ones128):
    @pl.kernel(
        out_type=jax.ShapeDtypeStruct((2, NP, 128), jnp.float32),
        mesh=_vec_mesh,
        scratch_types=[
            pltpu.VMEM((CHUNKS, 128), jnp.int32),
            pltpu.VMEM((128, 128), jnp.float32),
            pltpu.VMEM_SHARED((NP, 128), jnp.float32),
        ],
    )
    def count_kernel(dstm_hbm, z_hbm, ones_hbm, out_hbm, dst_v, ones_v, acc):
        c = lax.axis_index("c")
        s = lax.axis_index("s")
        r0 = s * ROWS_PER_SUB
        pltpu.sync_copy(z_hbm.at[pl.ds(r0, ROWS_PER_SUB)],
                        acc.at[pl.ds(r0, ROWS_PER_SUB)])
        pltpu.sync_copy(dstm_hbm.at[pl.ds(c * ER_PER_CORE + s * CHUNKS, CHUNKS)],
                        dst_v)
        pltpu.sync_copy(ones_hbm, ones_v)
        plsc.subcore_barrier()

        @pl.loop(0, CHUNKS)
        def _(j):
            pltpu.sync_copy(ones_v, acc.at[dst_v.at[j]], add=True)

        plsc.subcore_barrier()
        pltpu.sync_copy(acc.at[pl.ds(r0, ROWS_PER_SUB)],
                        out_hbm.at[c].at[pl.ds(r0, ROWS_PER_SUB)])

    return count_kernel(dstm, zeros128, ones128)


def _sc_aggregate(g, srcm, dstm, zeros128):
    @pl.kernel(
        out_type=jax.ShapeDtypeStruct((2, NP, D), jnp.float32),
        mesh=_vec_mesh,
        scratch_types=[
            pltpu.VMEM((CHUNKS // 2, 128), jnp.int32),
            pltpu.VMEM((CHUNKS // 2, 128), jnp.int32),
            pltpu.VMEM((64, D), jnp.float32),
            pltpu.VMEM((64, D), jnp.float32),
            pltpu.VMEM((64, D), jnp.float32),
            pltpu.VMEM((64, D), jnp.float32),
            pltpu.VMEM_SHARED((NP, D), jnp.float32),
            pltpu.SemaphoreType.DMA,
            pltpu.SemaphoreType.DMA,
            pltpu.SemaphoreType.DMA,
            pltpu.SemaphoreType.DMA,
        ],
    )
    def agg_kernel(g_hbm, srcm_hbm, dstm_hbm, z_hbm, out_hbm,
                   src_v, dst_v, rb0, rb1, rb2, rb3, acc, sm0, sm1, sm2, sm3):
        c = lax.axis_index("c")
        s = lax.axis_index("s")
        bufs = [rb0, rb1, rb2, rb3]
        sems = [sm0, sm1, sm2, sm3]
        r0 = s * ROWS_PER_SUB
        pltpu.sync_copy(z_hbm.at[pl.ds(r0, ROWS_PER_SUB)],
                        acc.at[pl.ds(r0, ROWS_PER_SUB)])
        plsc.subcore_barrier()

        def sidx(ref, q):
            return ref.at[q // 2, pl.ds(64 * (q % 2), 64)]

        def do_edges(base, stages):
            off = 0
            for nrows in stages:
                nsub = nrows * 2
                rsl = pl.ds(0, nrows)
                pltpu.sync_copy(srcm_hbm.at[pl.ds(base + off, nrows)],
                                src_v.at[rsl])
                pltpu.sync_copy(dstm_hbm.at[pl.ds(base + off, nrows)],
                                dst_v.at[rsl])
                for u in range(3):
                    pltpu.async_copy(g_hbm.at[sidx(src_v, u)], bufs[u], sems[u])

                @pl.loop(0, nsub // 4)
                def _(k):
                    for u in range(4):
                        q = k * 4 + u
                        pltpu.make_async_copy(g_hbm.at[sidx(src_v, q)],
                                              bufs[u], sems[u]).wait()

                        @pl.when(q + 3 < nsub)
                        def _():
                            pltpu.async_copy(g_hbm.at[sidx(src_v, q + 3)],
                                             bufs[(u + 3) % 4],
                                             sems[(u + 3) % 4])

                        pltpu.sync_copy(bufs[u], acc.at[sidx(dst_v, q)],
                                        add=True)

                off += nrows

        for k in range(5):
            blk = (s * 10 + k * 2) * 16
            do_edges(blk + c * 16, [16])

        plsc.subcore_barrier()
        pltpu.sync_copy(acc.at[pl.ds(r0, ROWS_PER_SUB)],
                        out_hbm.at[c].at[pl.ds(r0, ROWS_PER_SUB)])

    return agg_kernel(g, srcm, dstm, zeros128)


def _mm_body(x_ref, w_ref, o_ref):
    o_ref[...] = jnp.dot(x_ref[...], w_ref[...],
                         preferred_element_type=jnp.float32)


def _tc_matmul(x, w):
    return pl.pallas_call(
        _mm_body,
        out_shape=jax.ShapeDtypeStruct((NP, D), jnp.float32),
    )(x, w)


def _scale_body(c0_ref, c1_ref, h_ref, g_ref, dinv_ref):
    deg = c0_ref[:, 0:1] + c1_ref[:, 0:1] + 1.0
    rows = lax.broadcasted_iota(jnp.int32, (NP, 1), 0)
    dinv = jnp.where(rows < N, lax.rsqrt(deg), 0.0)
    dinv_ref[...] = dinv
    g_ref[...] = h_ref[...] * dinv


def _tc_scale(cnt0, cnt1, h):
    return pl.pallas_call(
        _scale_body,
        out_shape=(jax.ShapeDtypeStruct((NP, D), jnp.float32),
                   jax.ShapeDtypeStruct((NP, 1), jnp.float32)),
    )(cnt0, cnt1, h)


def _layer2_body(sa_ref, sb_ref, g1_ref, dinv_ref, b1_ref, w2_ref, g2_ref):
    dinv = dinv_ref[...]
    t = dinv * (sa_ref[...] + sb_ref[...] + g1_ref[...]) + b1_ref[...]
    z = jnp.maximum(t, 0.0)
    h2 = jnp.dot(z, w2_ref[...], preferred_element_type=jnp.float32)
    g2_ref[...] = h2 * dinv


def _tc_layer2(sa, sb, g1, dinv, b1, w2):
    return pl.pallas_call(
        _layer2_body,
        out_shape=jax.ShapeDtypeStruct((NP, D), jnp.float32),
    )(sa, sb, g1, dinv, b1, w2)


def _final_body(sa_ref, sb_ref, g2_ref, dinv_ref, b2_ref, o_ref):
    t = dinv_ref[...] * (sa_ref[...] + sb_ref[...] + g2_ref[...]) + b2_ref[...]
    m = jnp.max(t, axis=1, keepdims=True)
    lse = jnp.log(jnp.sum(jnp.exp(t - m), axis=1, keepdims=True)) + m
    o_ref[...] = t - lse


def _tc_final(sa, sb, g2, dinv, b2):
    return pl.pallas_call(
        _final_body,
        out_shape=jax.ShapeDtypeStruct((NP, D), jnp.float32),
    )(sa, sb, g2, dinv, b2)


def kernel(x, edge_index, W1, b1, W2, b2):
    f32 = jnp.float32
    xp = jnp.pad(x, ((0, NP - N), (0, 0)))
    pad = jnp.full((EP - E,), N, dtype=jnp.int32)
    srcm = jnp.concatenate([edge_index[0].astype(jnp.int32), pad]).reshape(ER, 128)
    dstm = jnp.concatenate([edge_index[1].astype(jnp.int32), pad]).reshape(ER, 128)
    zeros128 = jnp.zeros((NP, D), f32)
    ones128 = jnp.ones((128, 128), f32)

    cnt = _sc_count(dstm, zeros128, ones128)
    h1 = _tc_matmul(xp, W1)
    g1, dinv = _tc_scale(cnt[0], cnt[1], h1)
    s1 = _sc_aggregate(g1, srcm, dstm, zeros128)
    g2 = _tc_layer2(s1[0], s1[1], g1, dinv, b1.reshape(1, D), W2)
    s2 = _sc_aggregate(g2, srcm, dstm, zeros128)
    outp = _tc_final(s2[0], s2[1], g2, dinv, b2.reshape(1, D))
    return outp[:N]


# speedup vs baseline: 1.1993x; 1.0004x over previous
import jax
import jax.numpy as jnp
from jax import lax
from jax.experimental import pallas as pl
from jax.experimental.pallas import tpu as pltpu
from jax.experimental.pallas import tpu_sc as plsc

N = 10000
E = 320000
D = 128

NP = 10112
ROWS_PER_SUB = 632
EP = 327680
ER = EP // 128
CHUNKS = 80
ER_PER_CORE = ER // 2
SLOW_CORE = 1
SLOW_ROWS = 32
FAST_ROWS = 128

_vec_mesh = plsc.VectorSubcoreMesh(core_axis_name="c", subcore_axis_name="s")


def _sc_count(dstm, zeros128, ones128):
    @pl.kernel(
        out_type=jax.ShapeDtypeStruct((2, NP, 128), jnp.float32),
        mesh=_vec_mesh,
        scratch_types=[
            pltpu.VMEM((CHUNKS, 128), jnp.int32),
            pltpu.VMEM((128, 128), jnp.float32),
            pltpu.VMEM_SHARED((NP, 128), jnp.float32),
        ],
    )
    def count_kernel(dstm_hbm, z_hbm, ones_hbm, out_hbm, dst_v, ones_v, acc):
        c = lax.axis_index("c")
        s = lax.axis_index("s")
        r0 = s * ROWS_PER_SUB
        pltpu.sync_copy(z_hbm.at[pl.ds(r0, ROWS_PER_SUB)],
                        acc.at[pl.ds(r0, ROWS_PER_SUB)])
        pltpu.sync_copy(dstm_hbm.at[pl.ds(c * ER_PER_CORE + s * CHUNKS, CHUNKS)],
                        dst_v)
        pltpu.sync_copy(ones_hbm, ones_v)
        plsc.subcore_barrier()

        @pl.loop(0, CHUNKS)
        def _(j):
            pltpu.sync_copy(ones_v, acc.at[dst_v.at[j]], add=True)

        plsc.subcore_barrier()
        pltpu.sync_copy(acc.at[pl.ds(r0, ROWS_PER_SUB)],
                        out_hbm.at[c].at[pl.ds(r0, ROWS_PER_SUB)])

    return count_kernel(dstm, zeros128, ones128)


def _sc_aggregate(g, srcm, dstm, zeros128):
    @pl.kernel(
        out_type=jax.ShapeDtypeStruct((2, NP, D), jnp.float32),
        mesh=_vec_mesh,
        scratch_types=[
            pltpu.VMEM((CHUNKS // 2, 128), jnp.int32),
            pltpu.VMEM((CHUNKS // 2, 128), jnp.int32),
            pltpu.VMEM((64, D), jnp.float32),
            pltpu.VMEM((64, D), jnp.float32),
            pltpu.VMEM((64, D), jnp.float32),
            pltpu.VMEM((64, D), jnp.float32),
            pltpu.VMEM_SHARED((NP, D), jnp.float32),
            pltpu.SemaphoreType.DMA,
            pltpu.SemaphoreType.DMA,
            pltpu.SemaphoreType.DMA,
            pltpu.SemaphoreType.DMA,
        ],
    )
    def agg_kernel(g_hbm, srcm_hbm, dstm_hbm, z_hbm, out_hbm,
                   src_v, dst_v, rb0, rb1, rb2, rb3, acc, sm0, sm1, sm2, sm3):
        c = lax.axis_index("c")
        s = lax.axis_index("s")
        bufs = [rb0, rb1, rb2, rb3]
        sems = [sm0, sm1, sm2, sm3]
        r0 = s * ROWS_PER_SUB
        pltpu.sync_copy(z_hbm.at[pl.ds(r0, ROWS_PER_SUB)],
                        acc.at[pl.ds(r0, ROWS_PER_SUB)])
        plsc.subcore_barrier()

        def sidx(ref, q):
            return ref.at[q // 2, pl.ds(64 * (q % 2), 64)]

        def do_edges(base, stages):
            off = 0
            for nrows in stages:
                nsub = nrows * 2
                rsl = pl.ds(0, nrows)
                pltpu.sync_copy(srcm_hbm.at[pl.ds(base + off, nrows)],
                                src_v.at[rsl])
                pltpu.sync_copy(dstm_hbm.at[pl.ds(base + off, nrows)],
                                dst_v.at[rsl])
                for u in range(3):
                    pltpu.async_copy(g_hbm.at[sidx(src_v, u)], bufs[u], sems[u])

                @pl.loop(0, nsub // 4)
                def _(k):
                    for u in range(4):
                        q = k * 4 + u
                        pltpu.make_async_copy(g_hbm.at[sidx(src_v, q)],
                                              bufs[u], sems[u]).wait()

                        @pl.when(q + 3 < nsub)
                        def _():
                            pltpu.async_copy(g_hbm.at[sidx(src_v, q + 3)],
                                             bufs[(u + 3) % 4],
                                             sems[(u + 3) % 4])

                        pltpu.sync_copy(bufs[u], acc.at[sidx(dst_v, q)],
                                        add=True)

                off += nrows

        for k in range(5):
            blk = (s * 10 + k * 2) * 16
            do_edges(blk + c * 16, [16])

        plsc.subcore_barrier()
        pltpu.sync_copy(acc.at[pl.ds(r0, ROWS_PER_SUB)],
                        out_hbm.at[c].at[pl.ds(r0, ROWS_PER_SUB)])

    return agg_kernel(g, srcm, dstm, zeros128)


def _mm_body(x_ref, w_ref, o_ref):
    o_ref[...] = jnp.dot(x_ref[...], w_ref[...],
                         preferred_element_type=jnp.float32)


def _tc_matmul(x, w):
    return pl.pallas_call(
        _mm_body,
        out_shape=jax.ShapeDtypeStruct((NP, D), jnp.float32),
    )(x, w)


def _scale_body(c0_ref, c1_ref, h_ref, g_ref, dinv_ref):
    deg = c0_ref[:, 0:1] + c1_ref[:, 0:1] + 1.0
    rows = lax.broadcasted_iota(jnp.int32, (NP, 1), 0)
    dinv = jnp.where(rows < N, lax.rsqrt(deg), 0.0)
    dinv_ref[...] = dinv
    g_ref[...] = h_ref[...] * dinv


def _tc_scale(cnt0, cnt1, h):
    return pl.pallas_call(
        _scale_body,
        out_shape=(jax.ShapeDtypeStruct((NP, D), jnp.float32),
                   jax.ShapeDtypeStruct((NP, 1), jnp.float32)),
    )(cnt0, cnt1, h)


def _layer2_body(sa_ref, sb_ref, g1_ref, dinv_ref, b1_ref, w2_ref, g2_ref):
    dinv = dinv_ref[...]
    t = dinv * (sa_ref[...] + sb_ref[...] + g1_ref[...]) + b1_ref[...]
    z = jnp.maximum(t, 0.0)
    h2 = jnp.dot(z, w2_ref[...], preferred_element_type=jnp.float32)
    g2_ref[...] = h2 * dinv


def _tc_layer2(sa, sb, g1, dinv, b1, w2):
    return pl.pallas_call(
        _layer2_body,
        out_shape=jax.ShapeDtypeStruct((NP, D), jnp.float32),
    )(sa, sb, g1, dinv, b1, w2)


def _final_body(sa_ref, sb_ref, g2_ref, dinv_ref, b2_ref, o_ref):
    t = dinv_ref[...] * (sa_ref[...] + sb_ref[...] + g2_ref[...]) + b2_ref[...]
    m = jnp.max(t, axis=1, keepdims=True)
    lse = jnp.log(jnp.sum(jnp.exp(t - m), axis=1, keepdims=True)) + m
    o_ref[...] = t - lse


def _tc_final(sa, sb, g2, dinv, b2):
    return pl.pallas_call(
        _final_body,
        out_shape=jax.ShapeDtypeStruct((NP, D), jnp.float32),
    )(sa, sb, g2, dinv, b2)


def kernel(x, edge_index, W1, b1, W2, b2):
    f32 = jnp.float32
    xp = jnp.pad(x, ((0, NP - N), (0, 0)))
    pad_src = jnp.full((EP - E,), N, dtype=jnp.int32)
    pad_dst = N + (jnp.arange(EP - E, dtype=jnp.int32) % 64)
    srcm = jnp.concatenate([edge_index[0].astype(jnp.int32), pad_src]).reshape(ER, 128)
    dstm = jnp.concatenate([edge_index[1].astype(jnp.int32), pad_dst]).reshape(ER, 128)
    zeros128 = jnp.zeros((NP, D), f32)
    ones128 = jnp.ones((128, 128), f32)

    cnt = _sc_count(dstm, zeros128, ones128)
    h1 = _tc_matmul(xp, W1)
    g1, dinv = _tc_scale(cnt[0], cnt[1], h1)
    s1 = _sc_aggregate(g1, srcm, dstm, zeros128)
    g2 = _tc_layer2(s1[0], s1[1], g1, dinv, b1.reshape(1, D), W2)
    s2 = _sc_aggregate(g2, srcm, dstm, zeros128)
    outp = _tc_final(s2[0], s2[1], g2, dinv, b2.reshape(1, D))
    return outp[:N]


# speedup vs baseline: 1.2330x; 1.0281x over previous
import jax
import jax.numpy as jnp
from jax import lax
from jax.experimental import pallas as pl
from jax.experimental.pallas import tpu as pltpu
from jax.experimental.pallas import tpu_sc as plsc

N = 10000
E = 320000
D = 128

NP = 10112
ROWS_PER_SUB = 632
EP = 327680
ER = EP // 128
CHUNKS = 80
ER_PER_CORE = ER // 2
SLOW_CORE = 1
SLOW_ROWS = 32
FAST_ROWS = 128

_vec_mesh = plsc.VectorSubcoreMesh(core_axis_name="c", subcore_axis_name="s")


def _sc_count(dstm, zeros128, ones128):
    @pl.kernel(
        out_type=jax.ShapeDtypeStruct((2, NP, 128), jnp.float32),
        mesh=_vec_mesh,
        scratch_types=[
            pltpu.VMEM((CHUNKS, 128), jnp.int32),
            pltpu.VMEM((128, 128), jnp.float32),
            pltpu.VMEM_SHARED((NP, 128), jnp.float32),
        ],
    )
    def count_kernel(dstm_hbm, z_hbm, ones_hbm, out_hbm, dst_v, ones_v, acc):
        c = lax.axis_index("c")
        s = lax.axis_index("s")
        r0 = s * ROWS_PER_SUB
        pltpu.sync_copy(z_hbm.at[pl.ds(r0, ROWS_PER_SUB)],
                        acc.at[pl.ds(r0, ROWS_PER_SUB)])
        pltpu.sync_copy(dstm_hbm.at[pl.ds(c * ER_PER_CORE + s * CHUNKS, CHUNKS)],
                        dst_v)
        pltpu.sync_copy(ones_hbm, ones_v)
        plsc.subcore_barrier()

        @pl.loop(0, CHUNKS)
        def _(j):
            pltpu.sync_copy(ones_v, acc.at[dst_v.at[j]], add=True)

        plsc.subcore_barrier()
        pltpu.sync_copy(acc.at[pl.ds(r0, ROWS_PER_SUB)],
                        out_hbm.at[c].at[pl.ds(r0, ROWS_PER_SUB)])

    return count_kernel(dstm, zeros128, ones128)


def _sc_aggregate(g, srcm, dstm, zeros128):
    @pl.kernel(
        out_type=jax.ShapeDtypeStruct((2, NP, D), jnp.float32),
        mesh=_vec_mesh,
        scratch_types=[
            pltpu.VMEM((CHUNKS // 2, 128), jnp.int32),
            pltpu.VMEM((CHUNKS // 2, 128), jnp.int32),
            pltpu.VMEM((64, D), jnp.float32),
            pltpu.VMEM((64, D), jnp.float32),
            pltpu.VMEM((64, D), jnp.float32),
            pltpu.VMEM((64, D), jnp.float32),
            pltpu.VMEM_SHARED((NP, D), jnp.float32),
            pltpu.SemaphoreType.DMA,
            pltpu.SemaphoreType.DMA,
            pltpu.SemaphoreType.DMA,
            pltpu.SemaphoreType.DMA,
        ],
    )
    def agg_kernel(g_hbm, srcm_hbm, dstm_hbm, z_hbm, out_hbm,
                   src_v, dst_v, rb0, rb1, rb2, rb3, acc, sm0, sm1, sm2, sm3):
        c = lax.axis_index("c")
        s = lax.axis_index("s")
        bufs = [rb0, rb1, rb2, rb3]
        sems = [sm0, sm1, sm2, sm3]
        r0 = s * ROWS_PER_SUB
        pltpu.sync_copy(z_hbm.at[pl.ds(r0, ROWS_PER_SUB)],
                        acc.at[pl.ds(r0, ROWS_PER_SUB)])
        plsc.subcore_barrier()

        def sidx(ref, q):
            return ref.at[q // 2, pl.ds(64 * (q % 2), 64)]

        def do_edges(base, stages):
            off = 0
            for nrows in stages:
                nsub = nrows * 2
                rsl = pl.ds(0, nrows)
                pltpu.sync_copy(srcm_hbm.at[pl.ds(base + off, nrows)],
                                src_v.at[rsl])
                pltpu.sync_copy(dstm_hbm.at[pl.ds(base + off, nrows)],
                                dst_v.at[rsl])
                for u in range(3):
                    pltpu.async_copy(g_hbm.at[sidx(src_v, u)], bufs[u], sems[u])

                @pl.loop(0, nsub // 4)
                def _(k):
                    for u in range(4):
                        q = k * 4 + u
                        pltpu.make_async_copy(g_hbm.at[sidx(src_v, q)],
                                              bufs[u], sems[u]).wait()

                        @pl.when(q + 3 < nsub)
                        def _():
                            pltpu.async_copy(g_hbm.at[sidx(src_v, q + 3)],
                                             bufs[(u + 3) % 4],
                                             sems[(u + 3) % 4])

                        pltpu.sync_copy(bufs[u], acc.at[sidx(dst_v, q)],
                                        add=True)

                off += nrows

        for k in range(5):
            blk = (s * 10 + k * 2) * 16
            do_edges(blk + c * 16, [16])

        plsc.subcore_barrier()
        pltpu.sync_copy(acc.at[pl.ds(r0, ROWS_PER_SUB)],
                        out_hbm.at[c].at[pl.ds(r0, ROWS_PER_SUB)])

    return agg_kernel(g, srcm, dstm, zeros128)


def _mm_body(x_ref, w_ref, o_ref):
    o_ref[...] = jnp.dot(x_ref[...], w_ref[...],
                         preferred_element_type=jnp.float32)


def _tc_matmul(x, w):
    return pl.pallas_call(
        _mm_body,
        out_shape=jax.ShapeDtypeStruct((NP, D), jnp.float32),
    )(x, w)


def _scale_body(cnt_ref, h_ref, g_ref, dinv_ref):
    deg = cnt_ref[0, :, 0:1] + cnt_ref[1, :, 0:1] + 1.0
    rows = lax.broadcasted_iota(jnp.int32, (NP, 1), 0)
    dinv = jnp.where(rows < N, lax.rsqrt(deg), 0.0)
    dinv_ref[...] = dinv
    g_ref[...] = h_ref[...] * dinv


def _tc_scale(cnt, h):
    return pl.pallas_call(
        _scale_body,
        out_shape=(jax.ShapeDtypeStruct((NP, D), jnp.float32),
                   jax.ShapeDtypeStruct((NP, 1), jnp.float32)),
    )(cnt, h)


def _layer2_body(s_ref, g1_ref, dinv_ref, b1_ref, w2_ref, g2_ref):
    dinv = dinv_ref[...]
    t = dinv * (s_ref[0] + s_ref[1] + g1_ref[...]) + b1_ref[...]
    z = jnp.maximum(t, 0.0)
    h2 = jnp.dot(z, w2_ref[...], preferred_element_type=jnp.float32)
    g2_ref[...] = h2 * dinv


def _tc_layer2(s, g1, dinv, b1, w2):
    return pl.pallas_call(
        _layer2_body,
        out_shape=jax.ShapeDtypeStruct((NP, D), jnp.float32),
    )(s, g1, dinv, b1, w2)


def _final_body(s_ref, g2_ref, dinv_ref, b2_ref, o_ref):
    t = (dinv_ref[0:N] * (s_ref[0, 0:N] + s_ref[1, 0:N] + g2_ref[0:N])
         + b2_ref[...])
    m = jnp.max(t, axis=1, keepdims=True)
    lse = jnp.log(jnp.sum(jnp.exp(t - m), axis=1, keepdims=True)) + m
    o_ref[...] = t - lse


def _tc_final(s, g2, dinv, b2):
    return pl.pallas_call(
        _final_body,
        out_shape=jax.ShapeDtypeStruct((N, D), jnp.float32),
    )(s, g2, dinv, b2)


def kernel(x, edge_index, W1, b1, W2, b2):
    f32 = jnp.float32
    xp = jnp.pad(x, ((0, NP - N), (0, 0)))
    pad_src = jnp.full((EP - E,), N, dtype=jnp.int32)
    pad_dst = N + (jnp.arange(EP - E, dtype=jnp.int32) % 64)
    srcm = jnp.concatenate([edge_index[0].astype(jnp.int32), pad_src]).reshape(ER, 128)
    dstm = jnp.concatenate([edge_index[1].astype(jnp.int32), pad_dst]).reshape(ER, 128)
    zeros128 = jnp.zeros((NP, D), f32)
    ones128 = jnp.ones((128, 128), f32)

    cnt = _sc_count(dstm, zeros128, ones128)
    h1 = _tc_matmul(xp, W1)
    g1, dinv = _tc_scale(cnt, h1)
    s1 = _sc_aggregate(g1, srcm, dstm, zeros128)
    g2 = _tc_layer2(s1, g1, dinv, b1.reshape(1, D), W2)
    s2 = _sc_aggregate(g2, srcm, dstm, zeros128)
    return _tc_final(s2, g2, dinv, b2.reshape(1, D))
